# Initial kernel scaffold; baseline (speedup 1.0000x reference)
import jax
import jax.numpy as jnp
from jax.experimental import pallas as pl


def kernel(x, edge_index, W0, b0, W1, b1, W2, b2, S0w, S0b, S1w, S1b, S2w, S2b, g0, be0, g1, be1):
    raise NotImplementedError("write your pallas kernel here")


# speedup vs baseline: 8.3240x; 8.3240x over previous
import functools

import jax
import jax.numpy as jnp
from jax import lax
from jax.experimental import pallas as pl
from jax.experimental.pallas import tpu as pltpu
from jax.experimental.pallas import tpu_sc as plsc

NC = 2
NS = 16
K = 128
ZB = 32
EPS = 1e-5


def _sc_mesh():
    return plsc.VectorSubcoreMesh(
        core_axis_name="c", subcore_axis_name="s", num_cores=NC, num_subcores=NS)


def _sc_degree(col_pad, n, nrows):
    dc = 128
    e_pad = col_pad.shape[0]
    e_per_s = e_pad // (NC * NS)
    n_chunks = e_per_s // K
    zone = nrows // NS

    def body(col_hbm, out_hbm, cidx, ones, zbuf, acc):
        cid = lax.axis_index("c")
        sid = lax.axis_index("s")

        def fill(i, _):
            for j in range(dc // 16):
                ones[lax.rem(i, K), pl.ds(j * 16, 16)] = jnp.full(
                    (16,), 1.0, jnp.float32)
                zbuf[lax.rem(i, ZB), pl.ds(j * 16, 16)] = jnp.zeros(
                    (16,), jnp.float32)
            return 0
        lax.fori_loop(0, K, fill, 0)

        def zero_acc(i, _):
            pltpu.sync_copy(zbuf, acc.at[pl.ds(sid * zone + i * ZB, ZB)])
            return 0
        lax.fori_loop(0, zone // ZB, zero_acc, 0)
        plsc.subcore_barrier()

        base = (cid * NS + sid) * e_per_s

        def chunk(t, _):
            b = pl.multiple_of(base + t * K, K)
            pltpu.sync_copy(col_hbm.at[pl.ds(b, K)], cidx)
            pltpu.sync_copy(ones, acc.at[cidx], add=True)
            return 0
        lax.fori_loop(0, n_chunks, chunk, 0)
        plsc.subcore_barrier()

        @pl.when(sid * zone + zone <= n)
        def _():
            pltpu.sync_copy(acc.at[pl.ds(sid * zone, zone)],
                            out_hbm.at[pl.ds(cid * n + sid * zone, zone)])

        @pl.when(jnp.logical_and(sid * zone < n, sid * zone + zone > n))
        def _():
            tail = n - (NS - 1) * zone
            pltpu.sync_copy(acc.at[pl.ds(sid * zone, tail)],
                            out_hbm.at[pl.ds(cid * n + sid * zone, tail)])

    return pl.kernel(
        body,
        out_type=jax.ShapeDtypeStruct((2 * n, dc), jnp.float32),
        mesh=_sc_mesh(),
        scratch_types=[
            pltpu.VMEM((K,), jnp.int32),
            pltpu.VMEM((K, dc), jnp.float32),
            pltpu.VMEM((ZB, dc), jnp.float32),
            pltpu.VMEM_SHARED((nrows, dc), jnp.float32),
        ],
    )(col_pad)


def _sc_agg(yp, row_pad, col_pad, n, mode, nrows):
    dc = 128
    e_pad = row_pad.shape[0]
    e_per_s = e_pad // (NC * NS) if mode == "edges" else e_pad // NS
    n_chunks = e_per_s // K
    zone = nrows // NS

    def body(yp_hbm, row_hbm, col_hbm, out_hbm, ridx, cidx, rows, zbuf, acc, sem):
        cid = lax.axis_index("c")
        sid = lax.axis_index("s")

        def fill(i, _):
            for j in range(dc // 16):
                zbuf[i, pl.ds(j * 16, 16)] = jnp.zeros((16,), jnp.float32)
            return 0
        lax.fori_loop(0, ZB, fill, 0)

        def zero_acc(i, _):
            pltpu.sync_copy(zbuf, acc.at[pl.ds(sid * zone + i * ZB, ZB)])
            return 0
        lax.fori_loop(0, zone // ZB, zero_acc, 0)
        plsc.subcore_barrier()

        if mode == "edges":
            base = (cid * NS + sid) * e_per_s
        else:
            base = sid * e_per_s

        def chunk(t, _):
            b = pl.multiple_of(base + t * K, K)
            pltpu.sync_copy(row_hbm.at[pl.ds(b, K)], ridx)
            pltpu.sync_copy(col_hbm.at[pl.ds(b, K)], cidx)
            if mode == "channels":
                off = cid * n
                for j in range(K // 16):
                    sl = pl.ds(j * 16, 16)
                    ridx[sl] = ridx[sl] + off
            pltpu.async_copy(yp_hbm.at[ridx], rows, sem).wait()
            pltpu.sync_copy(rows, acc.at[cidx], add=True)
            return 0
        lax.fori_loop(0, n_chunks, chunk, 0)
        plsc.subcore_barrier()

        @pl.when(sid * zone + zone <= n)
        def _():
            pltpu.sync_copy(acc.at[pl.ds(sid * zone, zone)],
                            out_hbm.at[pl.ds(cid * n + sid * zone, zone)])

        @pl.when(jnp.logical_and(sid * zone < n, sid * zone + zone > n))
        def _():
            tail = n - (NS - 1) * zone
            pltpu.sync_copy(acc.at[pl.ds(sid * zone, tail)],
                            out_hbm.at[pl.ds(cid * n + sid * zone, tail)])

    return pl.kernel(
        body,
        out_type=jax.ShapeDtypeStruct((2 * n, dc), jnp.float32),
        mesh=_sc_mesh(),
        scratch_types=[
            pltpu.VMEM((K,), jnp.int32),
            pltpu.VMEM((K,), jnp.int32),
            pltpu.VMEM((K, dc), jnp.float32),
            pltpu.VMEM((ZB, dc), jnp.float32),
            pltpu.VMEM_SHARED((nrows, dc), jnp.float32),
            pltpu.SemaphoreType.DMA,
        ],
    )(yp, row_pad, col_pad)


def _deg_specs(r, nb):
    return [pl.BlockSpec((r, 128), lambda *g: (g[-1], 0)),
            pl.BlockSpec((r, 128), lambda *g, _nb=nb: (_nb + g[-1], 0))]


def _dinv(dlo_ref, dhi_ref):
    return lax.rsqrt(dlo_ref[:, 0:1] + dhi_ref[:, 0:1] + 1.0)


def _tc_prep(src, degp, n, dc, r):
    nb = n // r
    w = src.shape[1]

    def body(s_ref, dlo_ref, dhi_ref, o_ref):
        o_ref[...] = s_ref[:, 0:dc] * _dinv(dlo_ref, dhi_ref)

    return pl.pallas_call(
        body,
        grid=(nb,),
        in_specs=[pl.BlockSpec((r, w), lambda i: (i, 0))] + _deg_specs(r, nb),
        out_specs=pl.BlockSpec((r, dc), lambda i: (i, 0)),
        out_shape=jax.ShapeDtypeStruct((n, dc), jnp.float32),
    )(src, degp, degp)


def _tc_layer(sp, xin, degp, wcat, bias, n, din, dout, r, combine,
              stats=None):
    nb = n // r
    bn = stats is not None

    def body(*refs):
        if bn:
            (slo, shi, x_ref, dlo_ref, dhi_ref, w_ref, b_ref, su_ref, sq_ref,
             g_ref, be_ref, o_ref, so_ref, qo_ref) = refs
        else:
            (slo, shi, x_ref, dlo_ref, dhi_ref, w_ref, b_ref,
             o_ref, so_ref, qo_ref) = refs
        i = pl.program_id(0)
        x = x_ref[...]
        if bn:
            mu = jnp.sum(su_ref[...], axis=0, keepdims=True) / n
            var = jnp.sum(sq_ref[...], axis=0, keepdims=True) / n - mu * mu
            alpha = g_ref[...] * lax.rsqrt(var + EPS)
            beta = be_ref[...] - mu * alpha
            x = jnp.maximum(x * alpha + beta, 0.0)
        dinv = _dinv(dlo_ref, dhi_ref)
        if combine == "add":
            s = slo[...] + shi[...]
        else:
            s = jnp.concatenate([slo[...], shi[...]], axis=1)
        agg = s * dinv + x * (dinv * dinv)
        cat = jnp.concatenate([agg, x], axis=1)
        out = jnp.dot(cat, w_ref[...], preferred_element_type=jnp.float32)
        out = out + b_ref[...]
        o_ref[...] = out

        @pl.when(i == 0)
        def _():
            so_ref[...] = jnp.zeros_like(so_ref)
            qo_ref[...] = jnp.zeros_like(qo_ref)
        so_ref[...] += out.reshape(r // 8, 8, dout).sum(axis=0)
        qo_ref[...] += (out * out).reshape(r // 8, 8, dout).sum(axis=0)

    in_specs = [pl.BlockSpec((r, 128), lambda i: (i, 0)),
                pl.BlockSpec((r, 128), lambda i, _nb=nb: (_nb + i, 0)),
                pl.BlockSpec((r, din), lambda i: (i, 0))] + \
        _deg_specs(r, nb) + \
        [pl.BlockSpec((2 * din, dout), lambda i: (0, 0)),
         pl.BlockSpec((1, dout), lambda i: (0, 0))]
    args = [sp, sp, xin, degp, degp, wcat, bias]
    if bn:
        in_specs += [pl.BlockSpec((8, din), lambda i: (0, 0)),
                     pl.BlockSpec((8, din), lambda i: (0, 0)),
                     pl.BlockSpec((1, din), lambda i: (0, 0)),
                     pl.BlockSpec((1, din), lambda i: (0, 0))]
        args += list(stats)
    return pl.pallas_call(
        body,
        grid=(nb,),
        in_specs=in_specs,
        out_specs=[pl.BlockSpec((r, dout), lambda i: (i, 0)),
                   pl.BlockSpec((8, dout), lambda i: (0, 0)),
                   pl.BlockSpec((8, dout), lambda i: (0, 0))],
        out_shape=[jax.ShapeDtypeStruct((n, dout), jnp.float32),
                   jax.ShapeDtypeStruct((8, dout), jnp.float32),
                   jax.ShapeDtypeStruct((8, dout), jnp.float32)],
    )(*args)


def _tc_bnprep(xin, su, sq, g, be, degp, n, d, r):
    nb = n // r
    dc = d // 2

    def body(x_ref, su_ref, sq_ref, g_ref, be_ref, dlo_ref, dhi_ref, o_ref):
        mu = jnp.sum(su_ref[...], axis=0, keepdims=True) / n
        var = jnp.sum(sq_ref[...], axis=0, keepdims=True) / n - mu * mu
        alpha = g_ref[...] * lax.rsqrt(var + EPS)
        beta = be_ref[...] - mu * alpha
        v = jnp.maximum(x_ref[...] * alpha + beta, 0.0)
        o_ref[...] = v * _dinv(dlo_ref, dhi_ref)

    return pl.pallas_call(
        body,
        grid=(2, nb),
        in_specs=[pl.BlockSpec((r, dc), lambda c, i: (i, c)),
                  pl.BlockSpec((8, dc), lambda c, i: (0, c)),
                  pl.BlockSpec((8, dc), lambda c, i: (0, c)),
                  pl.BlockSpec((1, dc), lambda c, i: (0, c)),
                  pl.BlockSpec((1, dc), lambda c, i: (0, c))] +
        _deg_specs(r, nb),
        out_specs=pl.BlockSpec((r, dc), lambda c, i, _nb=nb: (c * _nb + i, 0)),
        out_shape=jax.ShapeDtypeStruct((2 * n, dc), jnp.float32),
    )(xin, su, sq, g, be, degp, degp)


def _tc_bnmm(xin, su, sq, g, be, w, n, d, dout, r):
    nb = n // r

    def body(x_ref, su_ref, sq_ref, g_ref, be_ref, w_ref, o_ref):
        mu = jnp.sum(su_ref[...], axis=0, keepdims=True) / n
        var = jnp.sum(sq_ref[...], axis=0, keepdims=True) / n - mu * mu
        alpha = g_ref[...] * lax.rsqrt(var + EPS)
        beta = be_ref[...] - mu * alpha
        v = jnp.maximum(x_ref[...] * alpha + beta, 0.0)
        o_ref[...] = jnp.dot(v, w_ref[...], preferred_element_type=jnp.float32)

    return pl.pallas_call(
        body,
        grid=(nb,),
        in_specs=[pl.BlockSpec((r, d), lambda i: (i, 0)),
                  pl.BlockSpec((8, d), lambda i: (0, 0)),
                  pl.BlockSpec((8, d), lambda i: (0, 0)),
                  pl.BlockSpec((1, d), lambda i: (0, 0)),
                  pl.BlockSpec((1, d), lambda i: (0, 0)),
                  pl.BlockSpec((d, dout), lambda i: (0, 0))],
        out_specs=pl.BlockSpec((r, dout), lambda i: (i, 0)),
        out_shape=jax.ShapeDtypeStruct((n, dout), jnp.float32),
    )(xin, su, sq, g, be, w)


def _tc_out(sp, zcat, degp, bias, n, dout, r):
    nb = n // r

    def body(slo, shi, z_ref, dlo_ref, dhi_ref, b_ref, o_ref):
        dinv = _dinv(dlo_ref, dhi_ref)
        s = slo[...] + shi[...]
        z = z_ref[:, 0:dout]
        u = z_ref[:, dout:2 * dout]
        o_ref[...] = s * dinv + z * (dinv * dinv) + u + b_ref[...]

    return pl.pallas_call(
        body,
        grid=(nb,),
        in_specs=[pl.BlockSpec((r, dout), lambda i: (i, 0)),
                  pl.BlockSpec((r, dout), lambda i, _nb=nb: (_nb + i, 0)),
                  pl.BlockSpec((r, 2 * dout), lambda i: (i, 0))] +
        _deg_specs(r, nb) +
        [pl.BlockSpec((1, dout), lambda i: (0, 0))],
        out_specs=pl.BlockSpec((r, dout), lambda i: (i, 0)),
        out_shape=jax.ShapeDtypeStruct((n, dout), jnp.float32),
    )(sp, sp, zcat, degp, degp, bias)


def kernel(x, edge_index, W0, b0, W1, b1, W2, b2, S0w, S0b, S1w, S1b,
           S2w, S2b, g0, be0, g1, be1):
    n, din = x.shape
    dh = W0.shape[1]
    dout = W2.shape[1]
    e = edge_index.shape[1]
    r = 1000

    e_pad = -(-e // (NC * NS * K)) * (NC * NS * K)
    row = edge_index[0]
    col = edge_index[1]
    if e_pad > e:
        row = jnp.concatenate([row, jnp.zeros((e_pad - e,), row.dtype)])
        col = jnp.concatenate([col, jnp.full((e_pad - e,), n, col.dtype)])
    nrows = -(-(n + 1) // (NS * ZB)) * (NS * ZB)

    wcat0 = jnp.concatenate([W0, S0w], axis=0)
    wcat1 = jnp.concatenate([W1, S1w], axis=0)
    wcat2 = jnp.concatenate([W2, S2w], axis=1)
    bias0 = (b0 + S0b).reshape(1, -1)
    bias1 = (b1 + S1b).reshape(1, -1)
    bias2 = (b2 + S2b).reshape(1, -1)
    g0r, be0r = g0.reshape(1, -1), be0.reshape(1, -1)
    g1r, be1r = g1.reshape(1, -1), be1.reshape(1, -1)

    degp = _sc_degree(col, n, nrows)

    y0 = _tc_prep(x, degp, n, din, r)
    s0p = _sc_agg(y0, row, col, n, "edges", nrows)
    out0, su0, sq0 = _tc_layer(s0p, x, degp, wcat0, bias0, n, din, dh, r,
                               combine="add")

    y1p = _tc_bnprep(out0, su0, sq0, g0r, be0r, degp, n, dh, r)
    s1p = _sc_agg(y1p, row, col, n, "channels", nrows)
    h, su1, sq1 = _tc_layer(s1p, out0, degp, wcat1, bias1, n, dh, dh, r,
                            combine="concat", stats=(su0, sq0, g0r, be0r))

    zcat = _tc_bnmm(h, su1, sq1, g1r, be1r, wcat2, n, dh, 2 * dout, r)
    y2 = _tc_prep(zcat, degp, n, dout, r)
    s2p = _sc_agg(y2, row, col, n, "edges", nrows)
    out2 = _tc_out(s2p, zcat, degp, bias2, n, dout, r)
    return (h, out2)
